# baseline (speedup 1.0000x reference)
import jax
import jax.numpy as jnp
from jax.experimental import pallas as pl
from jax.experimental.pallas import tpu as pltpu

N_ROWS = 4096
D = 1024
K_ORIG = 81
K_OUT = 81
NBB = 324
BN = 512


def _fused_body(x_ref, wb_ref, worig_ref, wcls_ref, borig_ref, bcls_ref,
                bb_ref, t_ref, scores_ref, bbox_ref, mm_ref):
    j = pl.program_id(0)
    nb = pl.num_programs(0)
    xb = x_ref[...]

    ssq = jnp.sum(xb * xb, axis=1, keepdims=True)
    xl2 = xb / jnp.maximum(jnp.sqrt(ssq), 1e-12)

    yo = jnp.dot(xb, worig_ref[...], preferred_element_type=jnp.float32)
    yo = yo + borig_ref[...]
    yc = jnp.dot(xl2, wcls_ref[...], preferred_element_type=jnp.float32)
    yc = yc + bcls_ref[...]
    yb = jnp.dot(xb, wb_ref[...], preferred_element_type=jnp.float32)
    yb = yb + bb_ref[...]
    bbox_ref[...] = jnp.dot(yb, t_ref[...], preferred_element_type=jnp.float32,
                            precision=jax.lax.Precision.HIGHEST)

    m = jnp.max(yo, axis=1, keepdims=True)
    flag = (yo[:, 0:1] >= m).astype(jnp.float32)

    lane = jax.lax.broadcasted_iota(jnp.int32, yc.shape, 1)
    scores_ref[pl.ds(j * BN, BN), :] = jnp.where(lane == 0, flag, yc)

    bmin = jnp.min(jnp.where(lane == 0, jnp.inf, yc))
    bmax = jnp.max(jnp.where(lane == 0, -jnp.inf, yc))

    @pl.when(j == 0)
    def _():
        mm_ref[0] = bmin
        mm_ref[1] = bmax

    @pl.when(j > 0)
    def _():
        mm_ref[0] = jnp.minimum(mm_ref[0], bmin)
        mm_ref[1] = jnp.maximum(mm_ref[1], bmax)

    @pl.when(j == nb - 1)
    def _():
        minv = mm_ref[0]
        maxv = mm_ref[1]
        full = scores_ref[...]
        lane_full = jax.lax.broadcasted_iota(jnp.int32, full.shape, 1)
        fixed = jnp.where(full > 0.5, maxv, minv)
        scores_ref[...] = jnp.where(lane_full == 0, fixed, full)


def kernel(x, W_bbox, b_bbox, W_orig, b_orig, W_cls, b_cls):
    f32 = jnp.float32
    x = x.reshape(x.shape[0], -1).astype(f32)

    wcls = jnp.concatenate([jnp.zeros((D, 1), f32), W_cls.astype(f32)], axis=1)
    bcls = jnp.concatenate([jnp.zeros((1,), f32), b_cls.astype(f32)])[None, :]
    borig = b_orig.astype(f32)[None, :]
    wb = jnp.concatenate([W_bbox.astype(f32), jnp.zeros((D, 4), f32)], axis=1)
    bb = jnp.concatenate([b_bbox.astype(f32), jnp.zeros((4,), f32)])[None, :]
    t = (jnp.arange(324)[None, :] % 4 == jnp.arange(8)[:, None]).astype(f32)

    nb = N_ROWS // BN
    grid = (nb,)
    full = lambda shape: pl.BlockSpec(shape, lambda j: (0, 0))
    scores, bbox = pl.pallas_call(
        _fused_body,
        grid=grid,
        in_specs=[
            pl.BlockSpec((BN, D), lambda j: (j, 0)),
            full((D, 8)),
            full((D, K_ORIG)),
            full((D, K_OUT)),
            full((1, K_ORIG)),
            full((1, K_OUT)),
            full((1, 8)),
            full((8, NBB)),
        ],
        out_specs=[
            pl.BlockSpec((N_ROWS, K_OUT), lambda j: (0, 0)),
            pl.BlockSpec((BN, NBB), lambda j: (j, 0)),
        ],
        out_shape=[
            jax.ShapeDtypeStruct((N_ROWS, K_OUT), f32),
            jax.ShapeDtypeStruct((N_ROWS, NBB), f32),
        ],
        scratch_shapes=[pltpu.SMEM((2,), f32)],
        compiler_params=pltpu.CompilerParams(
            dimension_semantics=("arbitrary",)),
    )(x, wb, W_orig.astype(f32), wcls, borig, bcls, bb, t)
    return (scores, bbox)


# speedup vs baseline: 1.1169x; 1.1169x over previous
import numpy as np
import jax
import jax.numpy as jnp
from jax.experimental import pallas as pl
from jax.experimental.pallas import tpu as pltpu

N_ROWS = 4096
D = 1024
K_ORIG = 81
K_CLS = 80
K_OUT = 81
NBB = 324
BN = 512

_TILE = np.equal(np.arange(NBB)[None, :] % 4,
                 np.arange(8)[:, None]).astype(np.float32)


def _fused_body(x_ref, wb_ref, bb_ref, worig_ref, borig_ref, wcls_ref,
                bcls_ref, t_ref, scores_ref, bbox_ref, mm_ref):
    j = pl.program_id(0)
    nb = pl.num_programs(0)
    xb = x_ref[...]

    ssq = jnp.sum(xb * xb, axis=1, keepdims=True)
    xl2 = xb / jnp.maximum(jnp.sqrt(ssq), 1e-12)

    yo = jnp.dot(xb, worig_ref[...], preferred_element_type=jnp.float32)
    yo = yo + borig_ref[...]
    yc = jnp.dot(xl2, wcls_ref[...], preferred_element_type=jnp.float32)
    yc = yc + bcls_ref[...]
    yb = jnp.dot(xb, wb_ref[...], preferred_element_type=jnp.float32)
    yb = yb + bb_ref[...]
    yb8 = jnp.concatenate([yb, jnp.zeros_like(yb)], axis=1)
    bbox_ref[...] = jnp.dot(yb8, t_ref[...], preferred_element_type=jnp.float32,
                            precision=jax.lax.Precision.HIGHEST)

    m = jnp.max(yo, axis=1, keepdims=True)
    flag = (yo[:, 0:1] >= m).astype(jnp.float32)

    scores_ref[pl.ds(j * BN, BN), :] = jnp.concatenate([flag, yc], axis=1)

    bmin = jnp.min(yc)
    bmax = jnp.max(yc)

    @pl.when(j == 0)
    def _():
        mm_ref[0] = bmin
        mm_ref[1] = bmax

    @pl.when(j > 0)
    def _():
        mm_ref[0] = jnp.minimum(mm_ref[0], bmin)
        mm_ref[1] = jnp.maximum(mm_ref[1], bmax)

    @pl.when(j == nb - 1)
    def _():
        minv = mm_ref[0]
        maxv = mm_ref[1]
        full = scores_ref[...]
        lane_full = jax.lax.broadcasted_iota(jnp.int32, full.shape, 1)
        fixed = jnp.where(full > 0.5, maxv, minv)
        scores_ref[...] = jnp.where(lane_full == 0, fixed, full)


def kernel(x, W_bbox, b_bbox, W_orig, b_orig, W_cls, b_cls):
    f32 = jnp.float32
    x = x.reshape(x.shape[0], -1).astype(f32)

    nb = N_ROWS // BN
    full = lambda shape: pl.BlockSpec(shape, lambda j: (0, 0))
    row1 = lambda n: pl.BlockSpec((1, n), lambda j: (0, 0))
    scores, bbox = pl.pallas_call(
        _fused_body,
        grid=(nb,),
        in_specs=[
            pl.BlockSpec((BN, D), lambda j: (j, 0)),
            full((D, 4)),
            row1(4),
            full((D, K_ORIG)),
            row1(K_ORIG),
            full((D, K_CLS)),
            row1(K_CLS),
            full((8, NBB)),
        ],
        out_specs=[
            pl.BlockSpec((N_ROWS, K_OUT), lambda j: (0, 0)),
            pl.BlockSpec((BN, NBB), lambda j: (j, 0)),
        ],
        out_shape=[
            jax.ShapeDtypeStruct((N_ROWS, K_OUT), f32),
            jax.ShapeDtypeStruct((N_ROWS, NBB), f32),
        ],
        scratch_shapes=[pltpu.SMEM((2,), f32)],
        compiler_params=pltpu.CompilerParams(
            dimension_semantics=("arbitrary",)),
    )(x, W_bbox.astype(f32), b_bbox.astype(f32).reshape(1, 4),
      W_orig.astype(f32), b_orig.astype(f32).reshape(1, K_ORIG),
      W_cls.astype(f32), b_cls.astype(f32).reshape(1, K_CLS),
      jnp.asarray(_TILE))
    return (scores, bbox)


# speedup vs baseline: 1.2369x; 1.1074x over previous
import jax
import jax.numpy as jnp
from jax.experimental import pallas as pl
from jax.experimental.pallas import tpu as pltpu

N_ROWS = 4096
D = 1024
K_ORIG = 81
K_CLS = 80
K_OUT = 81
NBB = 324
BN = 1024

def _fused_body(x_ref, wb_ref, bb_ref, worig_ref, borig_ref, wcls_ref,
                bcls_ref, scores_ref, bbox_ref, mm_ref):
    j = pl.program_id(0)
    nb = pl.num_programs(0)
    xb = x_ref[...]

    ssq = jnp.sum(xb * xb, axis=1, keepdims=True)
    xl2 = xb / jnp.maximum(jnp.sqrt(ssq), 1e-12)

    yo = jnp.dot(xb, worig_ref[...], preferred_element_type=jnp.float32)
    yo = yo + borig_ref[...]
    yc = jnp.dot(xl2, wcls_ref[...], preferred_element_type=jnp.float32)
    yc = yc + bcls_ref[...]
    yb = jnp.dot(xb, wb_ref[...], preferred_element_type=jnp.float32)
    yb = yb + bb_ref[...]
    idx = jax.lax.broadcasted_iota(jnp.int32, (yb.shape[0], NBB), 1) % 4
    bbox_ref[...] = jnp.take_along_axis(yb, idx, axis=1)

    m = jnp.max(yo, axis=1, keepdims=True)
    flag = (yo[:, 0:1] >= m).astype(jnp.float32)

    scores_ref[pl.ds(j * BN, BN), :] = jnp.concatenate([flag, yc], axis=1)

    bmin = jnp.min(yc)
    bmax = jnp.max(yc)

    @pl.when(j == 0)
    def _():
        mm_ref[0] = bmin
        mm_ref[1] = bmax

    @pl.when(j > 0)
    def _():
        mm_ref[0] = jnp.minimum(mm_ref[0], bmin)
        mm_ref[1] = jnp.maximum(mm_ref[1], bmax)

    @pl.when(j == nb - 1)
    def _():
        minv = mm_ref[0]
        maxv = mm_ref[1]
        full = scores_ref[...]
        lane_full = jax.lax.broadcasted_iota(jnp.int32, full.shape, 1)
        fixed = jnp.where(full > 0.5, maxv, minv)
        scores_ref[...] = jnp.where(lane_full == 0, fixed, full)


def kernel(x, W_bbox, b_bbox, W_orig, b_orig, W_cls, b_cls):
    f32 = jnp.float32
    x = x.reshape(x.shape[0], -1).astype(f32)

    nb = N_ROWS // BN
    full = lambda shape: pl.BlockSpec(shape, lambda j: (0, 0))
    row1 = lambda n: pl.BlockSpec((1, n), lambda j: (0, 0))
    scores, bbox = pl.pallas_call(
        _fused_body,
        grid=(nb,),
        in_specs=[
            pl.BlockSpec((BN, D), lambda j: (j, 0)),
            full((D, 4)),
            row1(4),
            full((D, K_ORIG)),
            row1(K_ORIG),
            full((D, K_CLS)),
            row1(K_CLS),
        ],
        out_specs=[
            pl.BlockSpec((N_ROWS, K_OUT), lambda j: (0, 0)),
            pl.BlockSpec((BN, NBB), lambda j: (j, 0)),
        ],
        out_shape=[
            jax.ShapeDtypeStruct((N_ROWS, K_OUT), f32),
            jax.ShapeDtypeStruct((N_ROWS, NBB), f32),
        ],
        scratch_shapes=[pltpu.SMEM((2,), f32)],
        compiler_params=pltpu.CompilerParams(
            dimension_semantics=("arbitrary",)),
    )(x, W_bbox.astype(f32), b_bbox.astype(f32).reshape(1, 4),
      W_orig.astype(f32), b_orig.astype(f32).reshape(1, K_ORIG),
      W_cls.astype(f32), b_cls.astype(f32).reshape(1, K_CLS))
    return (scores, bbox)
